# baseline (speedup 1.0000x reference)
import functools

import jax
import jax.numpy as jnp
from jax import lax
from jax.experimental import pallas as pl
from jax.experimental.pallas import tpu as pltpu
from jax.experimental.pallas import tpu_sc as plsc

_NC, _NS, _L = 2, 16, 16
_NW = _NC * _NS


def _knn(xyz1_ref, xyz2t_ref, i1_ref, i2_ref, i3_ref, w1_ref, w2_ref,
         w3_ref):
    b = pl.program_id(0)
    S = xyz2t_ref.shape[1]
    blkN = xyz1_ref.shape[2]

    x1 = xyz1_ref[0]
    x2t = xyz2t_ref[0]
    D = jnp.zeros((S, blkN), jnp.float32)
    for k in range(3):
        diff = x2t[:, k:k + 1] - x1[k:k + 1, :]
        D = D + diff * diff

    INF = jnp.float32(jnp.inf)
    iota = lax.broadcasted_iota(jnp.int32, (S, blkN), 0)
    d1 = jnp.min(D, axis=0, keepdims=True)
    M1 = D == d1
    D1 = jnp.where(M1, INF, D)
    d2 = jnp.min(D1, axis=0, keepdims=True)
    M2 = D1 == d2
    D2 = jnp.where(M2, INF, D1)
    d3 = jnp.min(D2, axis=0, keepdims=True)
    M3 = D2 == d3

    gbase = b * S
    i1_ref[0] = jnp.min(jnp.where(M1, iota, S), axis=0, keepdims=True) + gbase
    i2_ref[0] = jnp.min(jnp.where(M2, iota, S), axis=0, keepdims=True) + gbase
    i3_ref[0] = jnp.min(jnp.where(M3, iota, S), axis=0, keepdims=True) + gbase

    r1 = 1.0 / (d1 + 1e-8)
    r2 = 1.0 / (d2 + 1e-8)
    r3 = 1.0 / (d3 + 1e-8)
    norm = r1 + r2 + r3
    w1_ref[0] = r1 / norm
    w2_ref[0] = r2 / norm
    w3_ref[0] = r3 / norm


def _sc_interp(i1_hbm, i2_hbm, i3_hbm, w1_hbm, w2_hbm, w3_hbm, table_hbm,
               out_hbm, i1v, i2v, i3v, w1v, w2v, w3v,
               r1a, r2a, r3a, acca, r1b, r2b, r3b, accb,
               gsa, gsb, wsa, wsb, *, total_q, c2):
    qt = total_q // _NW
    ch = r1a.shape[0]
    nch = qt // ch
    wid = lax.axis_index("s") * _NC + lax.axis_index("c")
    base = wid * qt
    tsl = pl.ds(base, qt)
    tsl16 = pl.ds(base * _L, qt * _L)

    pltpu.sync_copy(i1_hbm.at[tsl], i1v)
    pltpu.sync_copy(i2_hbm.at[tsl], i2v)
    pltpu.sync_copy(i3_hbm.at[tsl], i3v)
    pltpu.sync_copy(w1_hbm.at[tsl16], w1v)
    pltpu.sync_copy(w2_hbm.at[tsl16], w2v)
    pltpu.sync_copy(w3_hbm.at[tsl16], w3v)

    bufs = ((r1a, r2a, r3a, acca, gsa, wsa), (r1b, r2b, r3b, accb, gsb, wsb))

    def issue(c, st):
        r1, r2, r3, _, gs, _ = bufs[st]
        csl = pl.ds(c * ch, ch)
        pltpu.async_copy(table_hbm.at[i1v.at[csl]], r1, gs)
        pltpu.async_copy(table_hbm.at[i2v.at[csl]], r2, gs)
        pltpu.async_copy(table_hbm.at[i3v.at[csl]], r3, gs)

    def work(c, st, first):
        r1, r2, r3, acc, gs, ws = bufs[st]
        pltpu.make_async_copy(table_hbm.at[i1v.at[pl.ds(0, ch)]], r1, gs).wait()
        pltpu.make_async_copy(table_hbm.at[i2v.at[pl.ds(0, ch)]], r2, gs).wait()
        pltpu.make_async_copy(table_hbm.at[i3v.at[pl.ds(0, ch)]], r3, gs).wait()
        if not first:
            pltpu.make_async_copy(acc, out_hbm.at[pl.ds(0, ch)], ws).wait()

        @plsc.parallel_loop(0, ch, step=1, carry=jnp.int32(0))
        def q_loop(q, carry2):
            qs = pl.ds((c * ch + q) * _L, _L)
            wq1 = w1v[qs]
            wq2 = w2v[qs]
            wq3 = w3v[qs]
            for f in range(c2 // _L):
                fs = pl.ds(f * _L, _L)
                acc[q, fs] = (wq1 * r1[q, fs] + wq2 * r2[q, fs]
                              + wq3 * r3[q, fs])
            return carry2
        @pl.when(c + 2 < nch)
        def _():
            issue(c + 2, st)
        pltpu.async_copy(acc, out_hbm.at[pl.ds(base + c * ch, ch)], ws)

    issue(0, 0)
    issue(1, 1)
    for c in range(nch):
        work(c, c % 2, first=(c < 2))
    pltpu.make_async_copy(acca, out_hbm.at[pl.ds(0, ch)], wsa).wait()
    pltpu.make_async_copy(accb, out_hbm.at[pl.ds(0, ch)], wsb).wait()


def _mlp(p1_ref, interp_ref, w0_ref, b0_ref, g0_ref, be0_ref, w1_ref, b1_ref,
         g1_ref, be1_ref, out_ref, x0s, x1s, sums0, sums1, *, count, blkN,
         c1):
    p = pl.program_id(0)
    b = pl.program_id(1)
    col = pl.ds(b * blkN, blkN)

    @pl.when((p == 0) & (b == 0))
    def _():
        sums0[...] = jnp.zeros_like(sums0)
        sums1[...] = jnp.zeros_like(sums1)

    @pl.when(p == 0)
    def _phase0():
        x0 = jax.lax.dot_general(w0_ref[:, :c1], p1_ref[0],
                                 (((1,), (0,)), ((), ())),
                                 preferred_element_type=jnp.float32)
        x0 = x0 + jax.lax.dot_general(w0_ref[:, c1:], interp_ref[...],
                                      (((1,), (1,)), ((), ())),
                                      preferred_element_type=jnp.float32)
        x0 = x0 + b0_ref[...]
        x0s[:, col] = x0
        s = jnp.sum(x0, axis=1, keepdims=True)
        sq = jnp.sum(x0 * x0, axis=1, keepdims=True)
        sums0[...] += jnp.concatenate([s, sq], axis=1)

    @pl.when(p == 1)
    def _phase1():
        mean = sums0[:, 0:1] / count
        var = sums0[:, 1:2] / count - mean * mean
        scale = jax.lax.rsqrt(var + 1e-5) * g0_ref[...]
        xn = jnp.maximum((x0s[:, col] - mean) * scale + be0_ref[...], 0.0)
        x1 = jax.lax.dot_general(w1_ref[...], xn, (((1,), (0,)), ((), ())),
                                 preferred_element_type=jnp.float32)
        x1 = x1 + b1_ref[...]
        x1s[:, col] = x1
        s = jnp.sum(x1, axis=1, keepdims=True)
        sq = jnp.sum(x1 * x1, axis=1, keepdims=True)
        sums1[...] += jnp.concatenate([s, sq], axis=1)

    @pl.when(p == 2)
    def _phase2():
        mean = sums1[:, 0:1] / count
        var = sums1[:, 1:2] / count - mean * mean
        scale = jax.lax.rsqrt(var + 1e-5) * g1_ref[...]
        out_ref[0] = jnp.maximum((x1s[:, col] - mean) * scale + be1_ref[...],
                                 0.0)


def kernel(xyz1, xyz2, points1, points2, W0, b0, g0, be0, W1, b1, g1, be1):
    B, _, N = xyz1.shape
    S = xyz2.shape[2]
    C1 = points1.shape[1]
    C2 = points2.shape[1]
    O0 = W0.shape[0]
    O1 = W1.shape[0]
    count = float(B * N)
    TQ = B * N
    QT = TQ // _NW
    CH = 32

    xyz2t = jnp.transpose(xyz2, (0, 2, 1))
    b0c = b0.reshape(O0, 1)
    g0c = g0.reshape(O0, 1)
    be0c = be0.reshape(O0, 1)
    b1c = b1.reshape(O1, 1)
    g1c = g1.reshape(O1, 1)
    be1c = be1.reshape(O1, 1)

    mesh = plsc.VectorSubcoreMesh(core_axis_name="c", subcore_axis_name="s")
    BH = B // 4
    TQH = BH * N

    def knn_half(x1h, x2th):
        return pl.pallas_call(
            _knn,
            grid=(BH,),
            in_specs=[
                pl.BlockSpec((1, 3, N), lambda b: (b, 0, 0)),
                pl.BlockSpec((1, S, 3), lambda b: (b, 0, 0)),
            ],
            out_specs=[pl.BlockSpec((1, 1, N), lambda b: (b, 0, 0))] * 6,
            out_shape=[jax.ShapeDtypeStruct((BH, 1, N), jnp.int32)] * 3
            + [jax.ShapeDtypeStruct((BH, 1, N), jnp.float32)] * 3,
        )(x1h, x2th)

    def sc_half(outs, table_h):
        i1, i2, i3 = (o.reshape(TQH) for o in outs[:3])
        w1, w2, w3 = (jnp.broadcast_to(o.reshape(TQH, 1),
                                       (TQH, _L)).reshape(-1)
                      for o in outs[3:])
        return pl.kernel(
            functools.partial(_sc_interp, total_q=TQH, c2=C2),
            out_type=jax.ShapeDtypeStruct((TQH, C2), jnp.float32),
            mesh=mesh,
            scratch_types=(
                [pltpu.VMEM((TQH // _NW,), jnp.int32)] * 3
                + [pltpu.VMEM((TQH // _NW * _L,), jnp.float32)] * 3
                + [pltpu.VMEM((CH, C2), jnp.float32)] * 4
                + [pltpu.VMEM((CH, C2), jnp.float32)] * 4
                + [pltpu.SemaphoreType.DMA] * 4
            ),
        )(i1, i2, i3, w1, w2, w3, table_h)

    table = jnp.transpose(points2, (0, 2, 1)).reshape(B * S, C2)
    knns = [knn_half(xyz1[i * BH:(i + 1) * BH], xyz2t[i * BH:(i + 1) * BH])
            for i in range(B // BH)]
    interps = [sc_half(k, table[i * BH * S:(i + 1) * BH * S])
               for i, k in enumerate(knns)]
    interp = jnp.concatenate(interps, axis=0)

    out = pl.pallas_call(
        functools.partial(_mlp, count=count, blkN=N, c1=C1),
        grid=(3, B),
        in_specs=[
            pl.BlockSpec((1, C1, N), lambda p, b: (b * (p == 0), 0, 0)),
            pl.BlockSpec((N, C2), lambda p, b: (b * (p == 0), 0)),
            pl.BlockSpec((O0, C1 + C2), lambda p, b: (0, 0)),
            pl.BlockSpec((O0, 1), lambda p, b: (0, 0)),
            pl.BlockSpec((O0, 1), lambda p, b: (0, 0)),
            pl.BlockSpec((O0, 1), lambda p, b: (0, 0)),
            pl.BlockSpec((O1, O0), lambda p, b: (0, 0)),
            pl.BlockSpec((O1, 1), lambda p, b: (0, 0)),
            pl.BlockSpec((O1, 1), lambda p, b: (0, 0)),
            pl.BlockSpec((O1, 1), lambda p, b: (0, 0)),
        ],
        out_specs=pl.BlockSpec((1, O1, N),
                               lambda p, b: (b * (p == 2), 0, 0)),
        out_shape=jax.ShapeDtypeStruct((B, O1, N), jnp.float32),
        scratch_shapes=[
            pltpu.VMEM((O0, B * N), jnp.float32),
            pltpu.VMEM((O1, B * N), jnp.float32),
            pltpu.VMEM((O0, 2), jnp.float32),
            pltpu.VMEM((O1, 2), jnp.float32),
        ],
    )(points1, interp, W0, b0c, g0c, be0c, W1, b1c, g1c, be1c)

    return out


# speedup vs baseline: 1.0378x; 1.0378x over previous
import functools

import jax
import jax.numpy as jnp
from jax import lax
from jax.experimental import pallas as pl
from jax.experimental.pallas import tpu as pltpu
from jax.experimental.pallas import tpu_sc as plsc

_NC, _NS, _L = 2, 16, 16
_NW = _NC * _NS


def _knn(xyz1_ref, xyz2t_ref, i1_ref, i2_ref, i3_ref, w1_ref, w2_ref,
         w3_ref):
    b = pl.program_id(0)
    S = xyz2t_ref.shape[1]
    blkN = xyz1_ref.shape[2]

    x1 = xyz1_ref[0]
    x2t = xyz2t_ref[0]
    D = jnp.zeros((S, blkN), jnp.float32)
    for k in range(3):
        diff = x2t[:, k:k + 1] - x1[k:k + 1, :]
        D = D + diff * diff

    INF = jnp.float32(jnp.inf)
    iota = lax.broadcasted_iota(jnp.int32, (S, blkN), 0)
    d1 = jnp.min(D, axis=0, keepdims=True)
    M1 = D == d1
    D1 = jnp.where(M1, INF, D)
    d2 = jnp.min(D1, axis=0, keepdims=True)
    M2 = D1 == d2
    D2 = jnp.where(M2, INF, D1)
    d3 = jnp.min(D2, axis=0, keepdims=True)
    M3 = D2 == d3

    gbase = b * S
    i1_ref[0] = jnp.min(jnp.where(M1, iota, S), axis=0, keepdims=True) + gbase
    i2_ref[0] = jnp.min(jnp.where(M2, iota, S), axis=0, keepdims=True) + gbase
    i3_ref[0] = jnp.min(jnp.where(M3, iota, S), axis=0, keepdims=True) + gbase

    r1 = 1.0 / (d1 + 1e-8)
    r2 = 1.0 / (d2 + 1e-8)
    r3 = 1.0 / (d3 + 1e-8)
    norm = r1 + r2 + r3
    w1_ref[0] = r1 / norm
    w2_ref[0] = r2 / norm
    w3_ref[0] = r3 / norm


def _sc_interp(i1_hbm, i2_hbm, i3_hbm, w1_hbm, w2_hbm, w3_hbm, table_hbm,
               out_hbm, i1v, i2v, i3v, w1v, w2v, w3v,
               r1a, r2a, r3a, acca, r1b, r2b, r3b, accb,
               gsa, gsb, wsa, wsb, *, total_q, c2):
    qt = total_q // _NW
    ch = r1a.shape[0]
    nch = qt // ch
    wid = lax.axis_index("s") * _NC + lax.axis_index("c")
    base = wid * qt
    tsl = pl.ds(base, qt)
    tsl16 = pl.ds(base * _L, qt * _L)

    pltpu.sync_copy(i1_hbm.at[tsl], i1v)
    pltpu.sync_copy(i2_hbm.at[tsl], i2v)
    pltpu.sync_copy(i3_hbm.at[tsl], i3v)
    pltpu.sync_copy(w1_hbm.at[tsl16], w1v)
    pltpu.sync_copy(w2_hbm.at[tsl16], w2v)
    pltpu.sync_copy(w3_hbm.at[tsl16], w3v)

    bufs = ((r1a, r2a, r3a, acca, gsa, wsa), (r1b, r2b, r3b, accb, gsb, wsb))

    def issue(c, st):
        r1, r2, r3, _, gs, _ = bufs[st]
        csl = pl.ds(c * ch, ch)
        pltpu.async_copy(table_hbm.at[i1v.at[csl]], r1, gs)
        pltpu.async_copy(table_hbm.at[i2v.at[csl]], r2, gs)
        pltpu.async_copy(table_hbm.at[i3v.at[csl]], r3, gs)

    def work(c, st, first):
        r1, r2, r3, acc, gs, ws = bufs[st]
        pltpu.make_async_copy(table_hbm.at[i1v.at[pl.ds(0, ch)]], r1, gs).wait()
        pltpu.make_async_copy(table_hbm.at[i2v.at[pl.ds(0, ch)]], r2, gs).wait()
        pltpu.make_async_copy(table_hbm.at[i3v.at[pl.ds(0, ch)]], r3, gs).wait()
        if not first:
            pltpu.make_async_copy(acc, out_hbm.at[pl.ds(0, ch)], ws).wait()

        @plsc.parallel_loop(0, ch, step=1, carry=jnp.int32(0))
        def q_loop(q, carry2):
            qs = pl.ds((c * ch + q) * _L, _L)
            wq1 = w1v[qs]
            wq2 = w2v[qs]
            wq3 = w3v[qs]
            for f in range(c2 // _L):
                fs = pl.ds(f * _L, _L)
                acc[q, fs] = (wq1 * r1[q, fs] + wq2 * r2[q, fs]
                              + wq3 * r3[q, fs])
            return carry2
        @pl.when(c + 2 < nch)
        def _():
            issue(c + 2, st)
        pltpu.async_copy(acc, out_hbm.at[pl.ds(base + c * ch, ch)], ws)

    issue(0, 0)
    issue(1, 1)
    for c in range(nch):
        work(c, c % 2, first=(c < 2))
    pltpu.make_async_copy(acca, out_hbm.at[pl.ds(0, ch)], wsa).wait()
    pltpu.make_async_copy(accb, out_hbm.at[pl.ds(0, ch)], wsb).wait()


def _mlp(p1_ref, interp_ref, w0_ref, b0_ref, g0_ref, be0_ref, w1_ref, b1_ref,
         g1_ref, be1_ref, out_ref, x0s, x1s, sums0, sums1, *, count, blkN,
         c1):
    p = pl.program_id(0)
    b = pl.program_id(1)
    col = pl.ds(b * blkN, blkN)

    @pl.when((p == 0) & (b == 0))
    def _():
        sums0[...] = jnp.zeros_like(sums0)
        sums1[...] = jnp.zeros_like(sums1)

    @pl.when(p == 0)
    def _phase0():
        x0 = jax.lax.dot_general(w0_ref[:, :c1], p1_ref[0],
                                 (((1,), (0,)), ((), ())),
                                 preferred_element_type=jnp.float32)
        x0 = x0 + jax.lax.dot_general(w0_ref[:, c1:], interp_ref[...],
                                      (((1,), (1,)), ((), ())),
                                      preferred_element_type=jnp.float32)
        x0 = x0 + b0_ref[...]
        x0s[:, col] = x0
        s = jnp.sum(x0, axis=1, keepdims=True)
        sq = jnp.sum(x0 * x0, axis=1, keepdims=True)
        sums0[...] += jnp.concatenate([s, sq], axis=1)

    @pl.when(p == 1)
    def _phase1():
        mean = sums0[:, 0:1] / count
        var = sums0[:, 1:2] / count - mean * mean
        scale = jax.lax.rsqrt(var + 1e-5) * g0_ref[...]
        xn = jnp.maximum((x0s[:, col] - mean) * scale + be0_ref[...], 0.0)
        x1 = jax.lax.dot_general(w1_ref[...], xn, (((1,), (0,)), ((), ())),
                                 preferred_element_type=jnp.float32)
        x1 = x1 + b1_ref[...]
        x1s[:, col] = x1
        s = jnp.sum(x1, axis=1, keepdims=True)
        sq = jnp.sum(x1 * x1, axis=1, keepdims=True)
        sums1[...] += jnp.concatenate([s, sq], axis=1)

    @pl.when(p == 2)
    def _phase2():
        mean = sums1[:, 0:1] / count
        var = sums1[:, 1:2] / count - mean * mean
        scale = jax.lax.rsqrt(var + 1e-5) * g1_ref[...]
        out_ref[0] = jnp.maximum((x1s[:, col] - mean) * scale + be1_ref[...],
                                 0.0)


def kernel(xyz1, xyz2, points1, points2, W0, b0, g0, be0, W1, b1, g1, be1):
    B, _, N = xyz1.shape
    S = xyz2.shape[2]
    C1 = points1.shape[1]
    C2 = points2.shape[1]
    O0 = W0.shape[0]
    O1 = W1.shape[0]
    count = float(B * N)
    TQ = B * N
    QT = TQ // _NW
    CH = 32

    xyz2t = jnp.transpose(xyz2, (0, 2, 1))
    b0c = b0.reshape(O0, 1)
    g0c = g0.reshape(O0, 1)
    be0c = be0.reshape(O0, 1)
    b1c = b1.reshape(O1, 1)
    g1c = g1.reshape(O1, 1)
    be1c = be1.reshape(O1, 1)

    mesh = plsc.VectorSubcoreMesh(core_axis_name="c", subcore_axis_name="s")
    BH = B // 2
    TQH = BH * N

    def knn_half(x1h, x2th):
        return pl.pallas_call(
            _knn,
            grid=(BH,),
            in_specs=[
                pl.BlockSpec((1, 3, N), lambda b: (b, 0, 0)),
                pl.BlockSpec((1, S, 3), lambda b: (b, 0, 0)),
            ],
            out_specs=[pl.BlockSpec((1, 1, N), lambda b: (b, 0, 0))] * 6,
            out_shape=[jax.ShapeDtypeStruct((BH, 1, N), jnp.int32)] * 3
            + [jax.ShapeDtypeStruct((BH, 1, N), jnp.float32)] * 3,
        )(x1h, x2th)

    def sc_half(outs, table_h):
        i1, i2, i3 = (o.reshape(TQH) for o in outs[:3])
        w1, w2, w3 = (jnp.broadcast_to(o.reshape(TQH, 1),
                                       (TQH, _L)).reshape(-1)
                      for o in outs[3:])
        return pl.kernel(
            functools.partial(_sc_interp, total_q=TQH, c2=C2),
            out_type=jax.ShapeDtypeStruct((TQH, C2), jnp.float32),
            mesh=mesh,
            scratch_types=(
                [pltpu.VMEM((TQH // _NW,), jnp.int32)] * 3
                + [pltpu.VMEM((TQH // _NW * _L,), jnp.float32)] * 3
                + [pltpu.VMEM((CH, C2), jnp.float32)] * 4
                + [pltpu.VMEM((CH, C2), jnp.float32)] * 4
                + [pltpu.SemaphoreType.DMA] * 4
            ),
        )(i1, i2, i3, w1, w2, w3, table_h)

    table = jnp.transpose(points2, (0, 2, 1)).reshape(B * S, C2)
    knns = [knn_half(xyz1[i * BH:(i + 1) * BH], xyz2t[i * BH:(i + 1) * BH])
            for i in range(B // BH)]
    interps = [sc_half(k, table[i * BH * S:(i + 1) * BH * S])
               for i, k in enumerate(knns)]
    interp = jnp.concatenate(interps, axis=0)

    out = pl.pallas_call(
        functools.partial(_mlp, count=count, blkN=N, c1=C1),
        grid=(3, B),
        in_specs=[
            pl.BlockSpec((1, C1, N), lambda p, b: (b * (p == 0), 0, 0)),
            pl.BlockSpec((N, C2), lambda p, b: (b * (p == 0), 0)),
            pl.BlockSpec((O0, C1 + C2), lambda p, b: (0, 0)),
            pl.BlockSpec((O0, 1), lambda p, b: (0, 0)),
            pl.BlockSpec((O0, 1), lambda p, b: (0, 0)),
            pl.BlockSpec((O0, 1), lambda p, b: (0, 0)),
            pl.BlockSpec((O1, O0), lambda p, b: (0, 0)),
            pl.BlockSpec((O1, 1), lambda p, b: (0, 0)),
            pl.BlockSpec((O1, 1), lambda p, b: (0, 0)),
            pl.BlockSpec((O1, 1), lambda p, b: (0, 0)),
        ],
        out_specs=pl.BlockSpec((1, O1, N),
                               lambda p, b: (b * (p == 2), 0, 0)),
        out_shape=jax.ShapeDtypeStruct((B, O1, N), jnp.float32),
        scratch_shapes=[
            pltpu.VMEM((O0, B * N), jnp.float32),
            pltpu.VMEM((O1, B * N), jnp.float32),
            pltpu.VMEM((O0, 2), jnp.float32),
            pltpu.VMEM((O1, 2), jnp.float32),
        ],
    )(points1, interp, W0, b0c, g0c, be0c, W1, b1c, g1c, be1c)

    return out


# speedup vs baseline: 1.1022x; 1.0621x over previous
import functools

import jax
import jax.numpy as jnp
from jax import lax
from jax.experimental import pallas as pl
from jax.experimental.pallas import tpu as pltpu
from jax.experimental.pallas import tpu_sc as plsc

_NC, _NS, _L = 2, 16, 16
_NW = _NC * _NS


def _knn(xyz1_ref, xyz2t_ref, i1_ref, i2_ref, i3_ref, w1_ref, w2_ref,
         w3_ref):
    b = pl.program_id(0)
    S = xyz2t_ref.shape[1]
    blkN = xyz1_ref.shape[2]

    x1 = xyz1_ref[0]
    x2t = xyz2t_ref[0]
    D = jnp.zeros((S, blkN), jnp.float32)
    for k in range(3):
        diff = x2t[:, k:k + 1] - x1[k:k + 1, :]
        D = D + diff * diff

    INF = jnp.float32(jnp.inf)
    iota = lax.broadcasted_iota(jnp.int32, (S, blkN), 0)
    d1 = jnp.min(D, axis=0, keepdims=True)
    M1 = D == d1
    D1 = jnp.where(M1, INF, D)
    d2 = jnp.min(D1, axis=0, keepdims=True)
    M2 = D1 == d2
    D2 = jnp.where(M2, INF, D1)
    d3 = jnp.min(D2, axis=0, keepdims=True)
    M3 = D2 == d3

    gbase = b * S
    i1_ref[0] = jnp.min(jnp.where(M1, iota, S), axis=0, keepdims=True) + gbase
    i2_ref[0] = jnp.min(jnp.where(M2, iota, S), axis=0, keepdims=True) + gbase
    i3_ref[0] = jnp.min(jnp.where(M3, iota, S), axis=0, keepdims=True) + gbase

    r1 = 1.0 / (d1 + 1e-8)
    r2 = 1.0 / (d2 + 1e-8)
    r3 = 1.0 / (d3 + 1e-8)
    norm = r1 + r2 + r3
    w1_ref[0] = r1 / norm
    w2_ref[0] = r2 / norm
    w3_ref[0] = r3 / norm


def _sc_interp(i1_hbm, i2_hbm, i3_hbm, w1_hbm, w2_hbm, w3_hbm, table_hbm,
               out_hbm, i1v, i2v, i3v, w1v, w2v, w3v,
               r1a, r2a, r3a, acca, r1b, r2b, r3b, accb,
               gsa, gsb, wsa, wsb, *, total_q, c2):
    qt = total_q // _NW
    ch = r1a.shape[0]
    nch = qt // ch
    wid = lax.axis_index("s") * _NC + lax.axis_index("c")
    base = wid * qt
    tsl = pl.ds(base, qt)
    tsl16 = pl.ds(base * _L, qt * _L)

    pltpu.sync_copy(i1_hbm.at[tsl], i1v)
    pltpu.sync_copy(i2_hbm.at[tsl], i2v)
    pltpu.sync_copy(i3_hbm.at[tsl], i3v)
    pltpu.sync_copy(w1_hbm.at[tsl16], w1v)
    pltpu.sync_copy(w2_hbm.at[tsl16], w2v)
    pltpu.sync_copy(w3_hbm.at[tsl16], w3v)

    bufs = ((r1a, r2a, r3a, acca, gsa, wsa), (r1b, r2b, r3b, accb, gsb, wsb))

    def issue(c, st):
        r1, r2, r3, _, gs, _ = bufs[st]
        csl = pl.ds(c * ch, ch)
        pltpu.async_copy(table_hbm.at[i1v.at[csl]], r1, gs)
        pltpu.async_copy(table_hbm.at[i2v.at[csl]], r2, gs)
        pltpu.async_copy(table_hbm.at[i3v.at[csl]], r3, gs)

    def work(c, st, first):
        r1, r2, r3, acc, gs, ws = bufs[st]
        pltpu.make_async_copy(table_hbm.at[i1v.at[pl.ds(0, ch)]], r1, gs).wait()
        pltpu.make_async_copy(table_hbm.at[i2v.at[pl.ds(0, ch)]], r2, gs).wait()
        pltpu.make_async_copy(table_hbm.at[i3v.at[pl.ds(0, ch)]], r3, gs).wait()
        if not first:
            pltpu.make_async_copy(acc, out_hbm.at[pl.ds(0, ch)], ws).wait()

        @plsc.parallel_loop(0, ch, step=1, carry=jnp.int32(0))
        def q_loop(q, carry2):
            qs = pl.ds((c * ch + q) * _L, _L)
            wq1 = w1v[qs]
            wq2 = w2v[qs]
            wq3 = w3v[qs]
            for f in range(c2 // _L):
                fs = pl.ds(f * _L, _L)
                acc[q, fs] = (wq1 * r1[q, fs] + wq2 * r2[q, fs]
                              + wq3 * r3[q, fs])
            return carry2
        @pl.when(c + 2 < nch)
        def _():
            issue(c + 2, st)
        pltpu.async_copy(acc, out_hbm.at[pl.ds(base + c * ch, ch)], ws)

    issue(0, 0)
    issue(1, 1)
    for c in range(nch):
        work(c, c % 2, first=(c < 2))
    pltpu.make_async_copy(acca, out_hbm.at[pl.ds(0, ch)], wsa).wait()
    pltpu.make_async_copy(accb, out_hbm.at[pl.ds(0, ch)], wsb).wait()


def _mlp(p1_ref, interp_a_ref, interp_b_ref, w0_ref, b0_ref, g0_ref,
         be0_ref, w1_ref, b1_ref, g1_ref, be1_ref, out_ref, x0s, x1s, sums0,
         sums1, *, count, blkN, c1, bh):
    p = pl.program_id(0)
    b = pl.program_id(1)
    col = pl.ds(b * blkN, blkN)

    @pl.when((p == 0) & (b == 0))
    def _():
        sums0[...] = jnp.zeros_like(sums0)
        sums1[...] = jnp.zeros_like(sums1)

    @pl.when(p == 0)
    def _phase0():
        x0 = jax.lax.dot_general(w0_ref[:, :c1], p1_ref[0],
                                 (((1,), (0,)), ((), ())),
                                 preferred_element_type=jnp.float32)
        interp = jnp.where(b < bh, interp_a_ref[...], interp_b_ref[...])
        x0 = x0 + jax.lax.dot_general(w0_ref[:, c1:], interp,
                                      (((1,), (1,)), ((), ())),
                                      preferred_element_type=jnp.float32)
        x0 = x0 + b0_ref[...]
        x0s[:, col] = x0
        s = jnp.sum(x0, axis=1, keepdims=True)
        sq = jnp.sum(x0 * x0, axis=1, keepdims=True)
        sums0[...] += jnp.concatenate([s, sq], axis=1)

    @pl.when(p == 1)
    def _phase1():
        mean = sums0[:, 0:1] / count
        var = sums0[:, 1:2] / count - mean * mean
        scale = jax.lax.rsqrt(var + 1e-5) * g0_ref[...]
        xn = jnp.maximum((x0s[:, col] - mean) * scale + be0_ref[...], 0.0)
        x1 = jax.lax.dot_general(w1_ref[...], xn, (((1,), (0,)), ((), ())),
                                 preferred_element_type=jnp.float32)
        x1 = x1 + b1_ref[...]
        x1s[:, col] = x1
        s = jnp.sum(x1, axis=1, keepdims=True)
        sq = jnp.sum(x1 * x1, axis=1, keepdims=True)
        sums1[...] += jnp.concatenate([s, sq], axis=1)

    @pl.when(p == 2)
    def _phase2():
        mean = sums1[:, 0:1] / count
        var = sums1[:, 1:2] / count - mean * mean
        scale = jax.lax.rsqrt(var + 1e-5) * g1_ref[...]
        out_ref[0] = jnp.maximum((x1s[:, col] - mean) * scale + be1_ref[...],
                                 0.0)


def kernel(xyz1, xyz2, points1, points2, W0, b0, g0, be0, W1, b1, g1, be1):
    B, _, N = xyz1.shape
    S = xyz2.shape[2]
    C1 = points1.shape[1]
    C2 = points2.shape[1]
    O0 = W0.shape[0]
    O1 = W1.shape[0]
    count = float(B * N)
    TQ = B * N
    QT = TQ // _NW
    CH = 32

    xyz2t = jnp.transpose(xyz2, (0, 2, 1))
    b0c = b0.reshape(O0, 1)
    g0c = g0.reshape(O0, 1)
    be0c = be0.reshape(O0, 1)
    b1c = b1.reshape(O1, 1)
    g1c = g1.reshape(O1, 1)
    be1c = be1.reshape(O1, 1)

    mesh = plsc.VectorSubcoreMesh(core_axis_name="c", subcore_axis_name="s")
    BH = B // 2
    TQH = BH * N

    def knn_half(x1h, x2th):
        return pl.pallas_call(
            _knn,
            grid=(BH,),
            in_specs=[
                pl.BlockSpec((1, 3, N), lambda b: (b, 0, 0)),
                pl.BlockSpec((1, S, 3), lambda b: (b, 0, 0)),
            ],
            out_specs=[pl.BlockSpec((1, 1, N), lambda b: (b, 0, 0))] * 6,
            out_shape=[jax.ShapeDtypeStruct((BH, 1, N), jnp.int32)] * 3
            + [jax.ShapeDtypeStruct((BH, 1, N), jnp.float32)] * 3,
        )(x1h, x2th)

    def sc_half(outs, table_h):
        i1, i2, i3 = (o.reshape(TQH) for o in outs[:3])
        w1, w2, w3 = (jnp.broadcast_to(o.reshape(TQH, 1),
                                       (TQH, _L)).reshape(-1)
                      for o in outs[3:])
        return pl.kernel(
            functools.partial(_sc_interp, total_q=TQH, c2=C2),
            out_type=jax.ShapeDtypeStruct((TQH, C2), jnp.float32),
            mesh=mesh,
            scratch_types=(
                [pltpu.VMEM((TQH // _NW,), jnp.int32)] * 3
                + [pltpu.VMEM((TQH // _NW * _L,), jnp.float32)] * 3
                + [pltpu.VMEM((CH, C2), jnp.float32)] * 4
                + [pltpu.VMEM((CH, C2), jnp.float32)] * 4
                + [pltpu.SemaphoreType.DMA] * 4
            ),
        )(i1, i2, i3, w1, w2, w3, table_h)

    table = jnp.transpose(points2, (0, 2, 1)).reshape(B * S, C2)
    knns = [knn_half(xyz1[i * BH:(i + 1) * BH], xyz2t[i * BH:(i + 1) * BH])
            for i in range(B // BH)]
    interps = [sc_half(k, table[i * BH * S:(i + 1) * BH * S])
               for i, k in enumerate(knns)]

    out = pl.pallas_call(
        functools.partial(_mlp, count=count, blkN=N, c1=C1, bh=BH),
        grid=(3, B),
        in_specs=[
            pl.BlockSpec((1, C1, N), lambda p, b: (b * (p == 0), 0, 0)),
            pl.BlockSpec((N, C2),
                         lambda p, b: (b * (p == 0) * (b < BH), 0)),
            pl.BlockSpec((N, C2),
                         lambda p, b: ((b - BH) * (p == 0) * (b >= BH), 0)),
            pl.BlockSpec((O0, C1 + C2), lambda p, b: (0, 0)),
            pl.BlockSpec((O0, 1), lambda p, b: (0, 0)),
            pl.BlockSpec((O0, 1), lambda p, b: (0, 0)),
            pl.BlockSpec((O0, 1), lambda p, b: (0, 0)),
            pl.BlockSpec((O1, O0), lambda p, b: (0, 0)),
            pl.BlockSpec((O1, 1), lambda p, b: (0, 0)),
            pl.BlockSpec((O1, 1), lambda p, b: (0, 0)),
            pl.BlockSpec((O1, 1), lambda p, b: (0, 0)),
        ],
        out_specs=pl.BlockSpec((1, O1, N),
                               lambda p, b: (b * (p == 2), 0, 0)),
        out_shape=jax.ShapeDtypeStruct((B, O1, N), jnp.float32),
        scratch_shapes=[
            pltpu.VMEM((O0, B * N), jnp.float32),
            pltpu.VMEM((O1, B * N), jnp.float32),
            pltpu.VMEM((O0, 2), jnp.float32),
            pltpu.VMEM((O1, 2), jnp.float32),
        ],
    )(points1, interps[0], interps[1], W0, b0c, g0c, be0c, W1, b1c, g1c,
      be1c)

    return out


# speedup vs baseline: 1.1061x; 1.0035x over previous
import functools

import jax
import jax.numpy as jnp
from jax import lax
from jax.experimental import pallas as pl
from jax.experimental.pallas import tpu as pltpu
from jax.experimental.pallas import tpu_sc as plsc

_NC, _NS, _L = 2, 16, 16
_NW = _NC * _NS


def _knn(xyz1_ref, xyz2t_ref, i1_ref, i2_ref, i3_ref, w1_ref, w2_ref,
         w3_ref):
    b = pl.program_id(0)
    S = xyz2t_ref.shape[1]
    blkN = xyz1_ref.shape[2]

    x1 = xyz1_ref[0]
    x2t = xyz2t_ref[0]
    D = jnp.zeros((S, blkN), jnp.float32)
    for k in range(3):
        diff = x2t[:, k:k + 1] - x1[k:k + 1, :]
        D = D + diff * diff

    INF = jnp.float32(jnp.inf)
    iota = lax.broadcasted_iota(jnp.int32, (S, blkN), 0)
    d1 = jnp.min(D, axis=0, keepdims=True)
    M1 = D == d1
    D1 = jnp.where(M1, INF, D)
    d2 = jnp.min(D1, axis=0, keepdims=True)
    M2 = D1 == d2
    D2 = jnp.where(M2, INF, D1)
    d3 = jnp.min(D2, axis=0, keepdims=True)
    M3 = D2 == d3

    gbase = b * S
    i1_ref[0] = jnp.min(jnp.where(M1, iota, S), axis=0, keepdims=True) + gbase
    i2_ref[0] = jnp.min(jnp.where(M2, iota, S), axis=0, keepdims=True) + gbase
    i3_ref[0] = jnp.min(jnp.where(M3, iota, S), axis=0, keepdims=True) + gbase

    r1 = 1.0 / (d1 + 1e-8)
    r2 = 1.0 / (d2 + 1e-8)
    r3 = 1.0 / (d3 + 1e-8)
    norm = r1 + r2 + r3
    w1_ref[0] = r1 / norm
    w2_ref[0] = r2 / norm
    w3_ref[0] = r3 / norm


def _sc_interp(i1_hbm, i2_hbm, i3_hbm, w1_hbm, w2_hbm, w3_hbm, table_hbm,
               out_hbm, i1v, i2v, i3v, w1v, w2v, w3v,
               r1a, r2a, r3a, acca, r1b, r2b, r3b, accb,
               r1c, r2c, r3c, accc, gsa, gsb, gsc, wsa, wsb, wsc,
               *, total_q, c2):
    qt = total_q // _NW
    ch = r1a.shape[0]
    nch = qt // ch
    wid = lax.axis_index("s") * _NC + lax.axis_index("c")
    base = wid * qt
    tsl = pl.ds(base, qt)
    tsl16 = pl.ds(base * _L, qt * _L)

    pltpu.sync_copy(i1_hbm.at[tsl], i1v)
    pltpu.sync_copy(i2_hbm.at[tsl], i2v)
    pltpu.sync_copy(i3_hbm.at[tsl], i3v)
    pltpu.sync_copy(w1_hbm.at[tsl16], w1v)
    pltpu.sync_copy(w2_hbm.at[tsl16], w2v)
    pltpu.sync_copy(w3_hbm.at[tsl16], w3v)

    bufs = ((r1a, r2a, r3a, acca, gsa, wsa), (r1b, r2b, r3b, accb, gsb, wsb),
            (r1c, r2c, r3c, accc, gsc, wsc))
    nset = len(bufs)

    def issue(c, st):
        r1, r2, r3, _, gs, _ = bufs[st]
        csl = pl.ds(c * ch, ch)
        pltpu.async_copy(table_hbm.at[i1v.at[csl]], r1, gs)
        pltpu.async_copy(table_hbm.at[i2v.at[csl]], r2, gs)
        pltpu.async_copy(table_hbm.at[i3v.at[csl]], r3, gs)

    def work(c, st, first):
        r1, r2, r3, acc, gs, ws = bufs[st]
        pltpu.make_async_copy(table_hbm.at[i1v.at[pl.ds(0, ch)]], r1, gs).wait()
        pltpu.make_async_copy(table_hbm.at[i2v.at[pl.ds(0, ch)]], r2, gs).wait()
        pltpu.make_async_copy(table_hbm.at[i3v.at[pl.ds(0, ch)]], r3, gs).wait()
        if not first:
            pltpu.make_async_copy(acc, out_hbm.at[pl.ds(0, ch)], ws).wait()

        @plsc.parallel_loop(0, ch, step=1, carry=jnp.int32(0))
        def q_loop(q, carry2):
            qs = pl.ds((c * ch + q) * _L, _L)
            wq1 = w1v[qs]
            wq2 = w2v[qs]
            wq3 = w3v[qs]
            for f in range(c2 // _L):
                fs = pl.ds(f * _L, _L)
                acc[q, fs] = (wq1 * r1[q, fs] + wq2 * r2[q, fs]
                              + wq3 * r3[q, fs])
            return carry2
        @pl.when(c + nset < nch)
        def _():
            issue(c + nset, st)
        pltpu.async_copy(acc, out_hbm.at[pl.ds(base + c * ch, ch)], ws)

    for st in range(nset):
        issue(st, st)
    for c in range(nch):
        work(c, c % nset, first=(c < nset))
    pltpu.make_async_copy(acca, out_hbm.at[pl.ds(0, ch)], wsa).wait()
    pltpu.make_async_copy(accb, out_hbm.at[pl.ds(0, ch)], wsb).wait()
    pltpu.make_async_copy(accc, out_hbm.at[pl.ds(0, ch)], wsc).wait()


def _mlp(p1_ref, interp_a_ref, interp_b_ref, w0_ref, b0_ref, g0_ref,
         be0_ref, w1_ref, b1_ref, g1_ref, be1_ref, out_ref, x0s, x1s, sums0,
         sums1, *, count, blkN, c1, bh):
    p = pl.program_id(0)
    b = pl.program_id(1)
    col = pl.ds(b * blkN, blkN)

    @pl.when((p == 0) & (b == 0))
    def _():
        sums0[...] = jnp.zeros_like(sums0)
        sums1[...] = jnp.zeros_like(sums1)

    @pl.when(p == 0)
    def _phase0():
        x0 = jax.lax.dot_general(w0_ref[:, :c1], p1_ref[0],
                                 (((1,), (0,)), ((), ())),
                                 preferred_element_type=jnp.float32)
        interp = jnp.where(b < bh, interp_a_ref[...], interp_b_ref[...])
        x0 = x0 + jax.lax.dot_general(w0_ref[:, c1:], interp,
                                      (((1,), (1,)), ((), ())),
                                      preferred_element_type=jnp.float32)
        x0 = x0 + b0_ref[...]
        x0s[:, col] = x0
        s = jnp.sum(x0, axis=1, keepdims=True)
        sq = jnp.sum(x0 * x0, axis=1, keepdims=True)
        sums0[...] += jnp.concatenate([s, sq], axis=1)

    @pl.when(p == 1)
    def _phase1():
        mean = sums0[:, 0:1] / count
        var = sums0[:, 1:2] / count - mean * mean
        scale = jax.lax.rsqrt(var + 1e-5) * g0_ref[...]
        xn = jnp.maximum((x0s[:, col] - mean) * scale + be0_ref[...], 0.0)
        x1 = jax.lax.dot_general(w1_ref[...], xn, (((1,), (0,)), ((), ())),
                                 preferred_element_type=jnp.float32)
        x1 = x1 + b1_ref[...]
        x1s[:, col] = x1
        s = jnp.sum(x1, axis=1, keepdims=True)
        sq = jnp.sum(x1 * x1, axis=1, keepdims=True)
        sums1[...] += jnp.concatenate([s, sq], axis=1)

    @pl.when(p == 2)
    def _phase2():
        mean = sums1[:, 0:1] / count
        var = sums1[:, 1:2] / count - mean * mean
        scale = jax.lax.rsqrt(var + 1e-5) * g1_ref[...]
        out_ref[0] = jnp.maximum((x1s[:, col] - mean) * scale + be1_ref[...],
                                 0.0)


def kernel(xyz1, xyz2, points1, points2, W0, b0, g0, be0, W1, b1, g1, be1):
    B, _, N = xyz1.shape
    S = xyz2.shape[2]
    C1 = points1.shape[1]
    C2 = points2.shape[1]
    O0 = W0.shape[0]
    O1 = W1.shape[0]
    count = float(B * N)
    TQ = B * N
    QT = TQ // _NW
    CH = 32

    xyz2t = jnp.transpose(xyz2, (0, 2, 1))
    b0c = b0.reshape(O0, 1)
    g0c = g0.reshape(O0, 1)
    be0c = be0.reshape(O0, 1)
    b1c = b1.reshape(O1, 1)
    g1c = g1.reshape(O1, 1)
    be1c = be1.reshape(O1, 1)

    mesh = plsc.VectorSubcoreMesh(core_axis_name="c", subcore_axis_name="s")
    BH = B // 2
    TQH = BH * N

    def knn_half(x1h, x2th):
        return pl.pallas_call(
            _knn,
            grid=(BH,),
            in_specs=[
                pl.BlockSpec((1, 3, N), lambda b: (b, 0, 0)),
                pl.BlockSpec((1, S, 3), lambda b: (b, 0, 0)),
            ],
            out_specs=[pl.BlockSpec((1, 1, N), lambda b: (b, 0, 0))] * 6,
            out_shape=[jax.ShapeDtypeStruct((BH, 1, N), jnp.int32)] * 3
            + [jax.ShapeDtypeStruct((BH, 1, N), jnp.float32)] * 3,
        )(x1h, x2th)

    def sc_half(outs, table_h):
        i1, i2, i3 = (o.reshape(TQH) for o in outs[:3])
        w1, w2, w3 = (jnp.broadcast_to(o.reshape(TQH, 1),
                                       (TQH, _L)).reshape(-1)
                      for o in outs[3:])
        return pl.kernel(
            functools.partial(_sc_interp, total_q=TQH, c2=C2),
            out_type=jax.ShapeDtypeStruct((TQH, C2), jnp.float32),
            mesh=mesh,
            scratch_types=(
                [pltpu.VMEM((TQH // _NW,), jnp.int32)] * 3
                + [pltpu.VMEM((TQH // _NW * _L,), jnp.float32)] * 3
                + [pltpu.VMEM((CH, C2), jnp.float32)] * 12
                + [pltpu.SemaphoreType.DMA] * 6
            ),
        )(i1, i2, i3, w1, w2, w3, table_h)

    table = jnp.transpose(points2, (0, 2, 1)).reshape(B * S, C2)
    knns = [knn_half(xyz1[i * BH:(i + 1) * BH], xyz2t[i * BH:(i + 1) * BH])
            for i in range(B // BH)]
    interps = [sc_half(k, table[i * BH * S:(i + 1) * BH * S])
               for i, k in enumerate(knns)]

    out = pl.pallas_call(
        functools.partial(_mlp, count=count, blkN=N, c1=C1, bh=BH),
        grid=(3, B),
        in_specs=[
            pl.BlockSpec((1, C1, N), lambda p, b: (b * (p == 0), 0, 0)),
            pl.BlockSpec((N, C2),
                         lambda p, b: (b * (p == 0) * (b < BH), 0)),
            pl.BlockSpec((N, C2),
                         lambda p, b: ((b - BH) * (p == 0) * (b >= BH), 0)),
            pl.BlockSpec((O0, C1 + C2), lambda p, b: (0, 0)),
            pl.BlockSpec((O0, 1), lambda p, b: (0, 0)),
            pl.BlockSpec((O0, 1), lambda p, b: (0, 0)),
            pl.BlockSpec((O0, 1), lambda p, b: (0, 0)),
            pl.BlockSpec((O1, O0), lambda p, b: (0, 0)),
            pl.BlockSpec((O1, 1), lambda p, b: (0, 0)),
            pl.BlockSpec((O1, 1), lambda p, b: (0, 0)),
            pl.BlockSpec((O1, 1), lambda p, b: (0, 0)),
        ],
        out_specs=pl.BlockSpec((1, O1, N),
                               lambda p, b: (b * (p == 2), 0, 0)),
        out_shape=jax.ShapeDtypeStruct((B, O1, N), jnp.float32),
        scratch_shapes=[
            pltpu.VMEM((O0, B * N), jnp.float32),
            pltpu.VMEM((O1, B * N), jnp.float32),
            pltpu.VMEM((O0, 2), jnp.float32),
            pltpu.VMEM((O1, 2), jnp.float32),
        ],
    )(points1, interps[0], interps[1], W0, b0c, g0c, be0c, W1, b1c, g1c,
      be1c)

    return out
